# baseline (speedup 1.0000x reference)
import functools
import jax
import jax.numpy as jnp
from jax import lax
from jax.experimental import pallas as pl
from jax.experimental.pallas import tpu as pltpu
from jax.experimental.pallas import tpu_sc as plsc

NUM_CLASSES = 100000
DIM = 64
BATCH = 16384
NC = 2
NS = 16
NW = NC * NS
ROWS_PER_W = BATCH // NW
CHUNK = 64
NCHUNK = ROWS_PER_W // CHUNK
LANES = 16
CW = CHUNK * DIM


def _sc_body(y_hbm, df_hbm, ct_hbm, out_hbm, idx_v, df_v, ct_v, acc_v,
             gsem, dsem):
    wid = lax.axis_index("s") * NC + lax.axis_index("c")
    base = pl.multiple_of(wid * ROWS_PER_W, ROWS_PER_W)
    pltpu.sync_copy(y_hbm.at[pl.ds(base, ROWS_PER_W)], idx_v)

    def issue(c, b):
        off = pl.multiple_of((base + c * CHUNK) * DIM, CW)
        pltpu.async_copy(df_hbm.at[pl.ds(off, CW)], df_v.at[b], dsem)

        def issue_group(g, _):
            vec = idx_v[pl.ds(c * CHUNK + g * LANES, LANES)] * DIM
            goff = g * (LANES * DIM)
            for i in range(LANES):
                pltpu.async_copy(
                    ct_hbm.at[pl.ds(pl.multiple_of(vec[i], DIM), DIM)],
                    ct_v.at[b, pl.ds(goff + i * DIM, DIM)], gsem)
            return 0

        lax.fori_loop(0, CHUNK // LANES, issue_group, 0)

    def drain(b):
        pltpu.make_async_copy(df_hbm.at[pl.ds(0, CW)], ct_v.at[b],
                              gsem).wait()
        pltpu.make_async_copy(df_hbm.at[pl.ds(0, CW)], df_v.at[b],
                              dsem).wait()

    def compute(b, acc):
        def row_body(i, a):
            for cc in range(DIM // LANES):
                d = (df_v[b, pl.ds(i * DIM + cc * LANES, LANES)]
                     - ct_v[b, pl.ds(i * DIM + cc * LANES, LANES)])
                a = a + d * d
            return a

        return lax.fori_loop(0, CHUNK, row_body, acc)

    acc = jnp.zeros((LANES,), jnp.float32)
    issue(0, 0)
    for c in range(NCHUNK):
        b = c % 2
        if c + 1 < NCHUNK:
            issue(c + 1, 1 - b)
        drain(b)
        acc = compute(b, acc)

    acc_v[...] = acc
    pltpu.sync_copy(
        acc_v, out_hbm.at[pl.ds(pl.multiple_of(wid * LANES, LANES), LANES)])


_sc_call = pl.kernel(
    _sc_body,
    out_type=jax.ShapeDtypeStruct((NW * LANES,), jnp.float32),
    mesh=plsc.VectorSubcoreMesh(core_axis_name="c", subcore_axis_name="s"),
    compiler_params=pltpu.CompilerParams(use_tc_tiling_on_sc=True),
    scratch_types=[
        pltpu.VMEM((ROWS_PER_W,), jnp.int32),
        pltpu.VMEM((2, CW), jnp.float32),
        pltpu.VMEM((2, CW), jnp.float32),
        pltpu.VMEM((LANES,), jnp.float32),
        pltpu.SemaphoreType.DMA,
        pltpu.SemaphoreType.DMA,
    ],
)


@jax.jit
def kernel(y, deep_feat, centers):
    partials = _sc_call(y.astype(jnp.int32), deep_feat.reshape(-1),
                        centers.reshape(-1))
    return 0.5 * jnp.sqrt(jnp.sum(partials)) / BATCH


# speedup vs baseline: 2.2451x; 2.2451x over previous
import functools
import jax
import jax.numpy as jnp
from jax import lax
from jax.experimental import pallas as pl
from jax.experimental.pallas import tpu as pltpu
from jax.experimental.pallas import tpu_sc as plsc

NUM_CLASSES = 100000
DIM = 64
BATCH = 16384
NC = 2
NS = 16
NW = NC * NS
FEATS_PER_W = DIM // NW
LANES = 16
STRIP = 4096
NSTRIP = BATCH // STRIP
GROUPS = STRIP // (2 * LANES)


def _sc_body(y_hbm, dft_hbm, ctt_hbm, out_hbm, row_v, y_v, df_v, acc_v,
             rsem, ysem, dsem):
    wid = lax.axis_index("s") * NC + lax.axis_index("c")

    def strip_copies(c, s, b):
        yc = pltpu.async_copy(y_hbm.at[pl.ds(s * STRIP, STRIP)],
                              y_v.at[pl.ds(b * STRIP, STRIP)], ysem)
        dc = pltpu.async_copy(dft_hbm.at[c, pl.ds(s * STRIP, STRIP)],
                              df_v.at[pl.ds(b * STRIP, STRIP)], dsem)
        return yc, dc

    def strip_compute(b, acc):
        base = b * STRIP

        def group(k, accs):
            a0, a1 = accs
            pos = base + k * (2 * LANES)
            yv0 = y_v[pl.ds(pos, LANES)]
            yv1 = y_v[pl.ds(pos + LANES, LANES)]
            g0 = plsc.load_gather(row_v, [yv0])
            g1 = plsc.load_gather(row_v, [yv1])
            d0 = df_v[pl.ds(pos, LANES)] - g0
            d1 = df_v[pl.ds(pos + LANES, LANES)] - g1
            return a0 + d0 * d0, a1 + d1 * d1

        return lax.fori_loop(0, GROUPS, group, acc)

    acc = (jnp.zeros((LANES,), jnp.float32), jnp.zeros((LANES,), jnp.float32))
    for f in range(FEATS_PER_W):
        c = wid + NW * f
        rc = pltpu.async_copy(ctt_hbm.at[c], row_v, rsem)
        cps = strip_copies(c, 0, 0)
        rc.wait()
        for s in range(NSTRIP):
            b = s % 2
            nxt = None
            if s + 1 < NSTRIP:
                nxt = strip_copies(c, s + 1, 1 - b)
            for cp in cps:
                cp.wait()
            acc = strip_compute(b, acc)
            cps = nxt

    acc_v[...] = acc[0] + acc[1]
    pltpu.sync_copy(
        acc_v, out_hbm.at[pl.ds(pl.multiple_of(wid * LANES, LANES), LANES)])


_sc_call = pl.kernel(
    _sc_body,
    out_type=jax.ShapeDtypeStruct((NW * LANES,), jnp.float32),
    mesh=plsc.VectorSubcoreMesh(core_axis_name="c", subcore_axis_name="s"),
    compiler_params=pltpu.CompilerParams(use_tc_tiling_on_sc=True,
                                         needs_layout_passes=False),
    scratch_types=[
        pltpu.VMEM((NUM_CLASSES,), jnp.float32),
        pltpu.VMEM((2 * STRIP,), jnp.int32),
        pltpu.VMEM((2 * STRIP,), jnp.float32),
        pltpu.VMEM((LANES,), jnp.float32),
        pltpu.SemaphoreType.DMA,
        pltpu.SemaphoreType.DMA,
        pltpu.SemaphoreType.DMA,
    ],
)


@jax.jit
def kernel(y, deep_feat, centers):
    partials = _sc_call(y.astype(jnp.int32), deep_feat.T, centers.T)
    return 0.5 * jnp.sqrt(jnp.sum(partials)) / BATCH


# speedup vs baseline: 2.2700x; 1.0111x over previous
import functools
import jax
import jax.numpy as jnp
from jax import lax
from jax.experimental import pallas as pl
from jax.experimental.pallas import tpu as pltpu
from jax.experimental.pallas import tpu_sc as plsc

NUM_CLASSES = 100000
DIM = 64
BATCH = 16384
NC = 2
NS = 16
NW = NC * NS
FEATS_PER_W = DIM // NW
LANES = 16
STRIP = 4096
NSTRIP = BATCH // STRIP
GROUPS = STRIP // (2 * LANES)


def _sc_body(y_hbm, dft_hbm, ctt_hbm, out_hbm, row_v, y_v, df_v, acc_v,
             rsem, ysem, dsem):
    wid = lax.axis_index("s") * NC + lax.axis_index("c")

    def strip_copies(c, s, b):
        yc = pltpu.async_copy(y_hbm.at[pl.ds(s * STRIP, STRIP)],
                              y_v.at[pl.ds(b * STRIP, STRIP)], ysem)
        dc = pltpu.async_copy(dft_hbm.at[c, pl.ds(s * STRIP, STRIP)],
                              df_v.at[pl.ds(b * STRIP, STRIP)], dsem)
        return yc, dc

    def strip_compute(b, acc):
        base = b * STRIP

        @plsc.parallel_loop(base, base + STRIP, 4 * LANES, unroll=2,
                            carry=acc)
        def loop(pos, accs):
            out = []
            for u in range(4):
                p = pos + u * LANES
                yv = y_v[pl.ds(p, LANES)]
                g = plsc.load_gather(row_v, [yv])
                d = df_v[pl.ds(p, LANES)] - g
                out.append(accs[u] + d * d)
            return tuple(out)

        return loop

    acc = tuple(jnp.zeros((LANES,), jnp.float32) for _ in range(4))
    for f in range(FEATS_PER_W):
        c = wid + NW * f
        rc = pltpu.async_copy(ctt_hbm.at[c], row_v, rsem)
        cps = strip_copies(c, 0, 0)
        rc.wait()
        for s in range(NSTRIP):
            b = s % 2
            nxt = None
            if s + 1 < NSTRIP:
                nxt = strip_copies(c, s + 1, 1 - b)
            for cp in cps:
                cp.wait()
            acc = strip_compute(b, acc)
            cps = nxt

    acc_v[...] = (acc[0] + acc[1]) + (acc[2] + acc[3])
    pltpu.sync_copy(
        acc_v, out_hbm.at[pl.ds(pl.multiple_of(wid * LANES, LANES), LANES)])


_sc_call = pl.kernel(
    _sc_body,
    out_type=jax.ShapeDtypeStruct((NW * LANES,), jnp.float32),
    mesh=plsc.VectorSubcoreMesh(core_axis_name="c", subcore_axis_name="s"),
    compiler_params=pltpu.CompilerParams(use_tc_tiling_on_sc=True,
                                         needs_layout_passes=False,
                                         disable_bounds_checks=True,
                                         disable_semaphore_checks=True),
    scratch_types=[
        pltpu.VMEM((NUM_CLASSES,), jnp.float32),
        pltpu.VMEM((2 * STRIP,), jnp.int32),
        pltpu.VMEM((2 * STRIP,), jnp.float32),
        pltpu.VMEM((LANES,), jnp.float32),
        pltpu.SemaphoreType.DMA,
        pltpu.SemaphoreType.DMA,
        pltpu.SemaphoreType.DMA,
    ],
)


@jax.jit
def kernel(y, deep_feat, centers):
    partials = _sc_call(y.astype(jnp.int32), deep_feat.T, centers.T)
    return 0.5 * jnp.sqrt(jnp.sum(partials)) / BATCH
